# baseline (speedup 1.0000x reference)
import functools

import jax
import jax.numpy as jnp
from jax import lax
from jax.experimental import pallas as pl
from jax.experimental.pallas import tpu as pltpu
from jax.experimental.pallas import tpu_sc as plsc

BATCH = 16384
EMBED = 64

_info = plsc.get_sparse_core_info()
NC, NS, L = _info.num_cores, _info.num_subcores, _info.num_lanes
NW = NC * NS
B_PER_W = BATCH // NW
IDXW = 128
HALF = B_PER_W // 2
GROUPS = HALF // L


def _sc_kernel(users_hbm, items_hbm, ut_hbm, it_hbm, out_hbm,
               uidx_v, iidx_v, ugid_v, igid_v, urow_v, irow_v, out_v, sem):
    wid = lax.axis_index("s") * NC + lax.axis_index("c")
    base = wid * B_PER_W

    pltpu.sync_copy(users_hbm.at[wid], uidx_v)
    pltpu.sync_copy(items_hbm.at[wid], iidx_v)

    for j in range(B_PER_W // L):
        s = pl.ds(j * L, L)
        ugid_v[s] = lax.shift_right_logical(uidx_v[s], 1)
        igid_v[s] = lax.shift_right_logical(iidx_v[s], 1)

    iota = lax.iota(jnp.int32, L)

    for h in range(2):
        cps = []
        for c in range(HALF // IDXW):
            s = pl.ds(h * HALF + c * IDXW, IDXW)
            d = pl.ds(c * IDXW, IDXW)
            cps.append(pltpu.async_copy(ut_hbm.at[ugid_v.at[s]], urow_v.at[d], sem))
            cps.append(pltpu.async_copy(it_hbm.at[igid_v.at[s]], irow_v.at[d], sem))
        for cp in cps:
            cp.wait()

        def group_body(g, carry):
            pos = h * HALF + g * L
            ubase = lax.bitwise_and(uidx_v[pl.ds(pos, L)], 1) * EMBED
            ibase = lax.bitwise_and(iidx_v[pl.ds(pos, L)], 1) * EMBED
            row = g * L + iota
            acc = jnp.zeros((L,), jnp.float32)
            for d in range(EMBED):
                dv = lax.bitwise_and(iota + d, EMBED - 1)
                uvals = plsc.load_gather(urow_v, [row, ubase + dv])
                ivals = plsc.load_gather(irow_v, [row, ibase + dv])
                acc = acc + uvals * ivals
            out_v[pl.ds(pos, L)] = acc
            return carry

        lax.fori_loop(0, GROUPS, group_body, 0)

    pltpu.sync_copy(out_v, out_hbm.at[pl.ds(base, B_PER_W)])


def kernel(users, items, user_table, item_table):
    users_2d = users.reshape(NW, B_PER_W)
    items_2d = items.reshape(NW, B_PER_W)
    ut_packed = user_table.reshape(user_table.shape[0] // 2, 2 * EMBED)
    it_packed = item_table.reshape(item_table.shape[0] // 2, 2 * EMBED)

    run = functools.partial(
        pl.kernel,
        mesh=plsc.VectorSubcoreMesh(core_axis_name="c", subcore_axis_name="s"),
        out_type=jax.ShapeDtypeStruct((BATCH,), jnp.float32),
        scratch_types=[
            pltpu.VMEM((B_PER_W,), jnp.int32),
            pltpu.VMEM((B_PER_W,), jnp.int32),
            pltpu.VMEM((B_PER_W,), jnp.int32),
            pltpu.VMEM((B_PER_W,), jnp.int32),
            pltpu.VMEM((HALF, 2 * EMBED), jnp.float32),
            pltpu.VMEM((HALF, 2 * EMBED), jnp.float32),
            pltpu.VMEM((B_PER_W,), jnp.float32),
            pltpu.SemaphoreType.DMA,
        ],
        compiler_params=pltpu.CompilerParams(needs_layout_passes=False),
    )(_sc_kernel)
    return run(users_2d, items_2d, ut_packed, it_packed)


# speedup vs baseline: 2.7811x; 2.7811x over previous
import functools

import jax
import jax.numpy as jnp
from jax import lax
from jax.experimental import pallas as pl
from jax.experimental.pallas import tpu as pltpu
from jax.experimental.pallas import tpu_sc as plsc

BATCH = 16384
EMBED = 64

_info = plsc.get_sparse_core_info()
NC, NS, L = _info.num_cores, _info.num_subcores, _info.num_lanes
NW = NC * NS
B_PER_W = BATCH // NW
K = 4
ROUNDS = B_PER_W // K
SLABW = 128


def _sc_kernel(users_hbm, items_hbm, ut_hbm, it_hbm, out_hbm,
               uidx_v, iidx_v, uslab_v, islab_v, out_v, usem, isem):
    wid = lax.axis_index("s") * NC + lax.axis_index("c")
    base = wid * B_PER_W

    pltpu.sync_copy(users_hbm.at[wid], uidx_v)
    pltpu.sync_copy(items_hbm.at[wid], iidx_v)

    iota = lax.iota(jnp.int32, L)

    def bidx(ref, e):
        return plsc.load_gather(ref, [jnp.full((L,), e, jnp.int32)])

    def fire(e, b):
        u = bidx(uidx_v, e)[0]
        uoff = pl.multiple_of(lax.shift_right_logical(u, 7) * SLABW, SLABW)
        pltpu.async_copy(ut_hbm.at[:, pl.ds(uoff, SLABW)], uslab_v.at[b], usem.at[b])
        v = bidx(iidx_v, e)[0]
        voff = pl.multiple_of(lax.shift_right_logical(v, 7) * SLABW, SLABW)
        pltpu.async_copy(it_hbm.at[:, pl.ds(voff, SLABW)], islab_v.at[b], isem.at[b])

    for b in range(K):
        fire(b, b)

    def round_body(r, vec):
        lane0 = lax.rem(r, L // K) * K
        for b in range(K):
            e = r * K + b
            pltpu.make_async_copy(
                ut_hbm.at[:, pl.ds(0, SLABW)], uslab_v.at[b], usem.at[b]).wait()
            pltpu.make_async_copy(
                it_hbm.at[:, pl.ds(0, SLABW)], islab_v.at[b], isem.at[b]).wait()

            cu = lax.bitwise_and(bidx(uidx_v, e), SLABW - 1)
            ci = lax.bitwise_and(bidx(iidx_v, e), SLABW - 1)
            acc = jnp.zeros((L,), jnp.float32)
            for k in range(EMBED // L):
                rows = k * L + iota
                uv = plsc.load_gather(uslab_v.at[b], [rows, cu])
                iv = plsc.load_gather(islab_v.at[b], [rows, ci])
                acc = acc + uv * iv
            vec = jnp.where(iota == lane0 + b, jnp.sum(acc), vec)

            f = e + K
            @pl.when(f < B_PER_W)
            def _():
                fire(f, b)

        @pl.when(lane0 + K == L)
        def _():
            out_v[pl.ds((r // (L // K)) * L, L)] = vec
        return vec

    lax.fori_loop(0, ROUNDS, round_body, jnp.zeros((L,), jnp.float32))

    pltpu.sync_copy(out_v, out_hbm.at[pl.ds(base, B_PER_W)])


def kernel(users, items, user_table, item_table):
    users_2d = users.reshape(NW, B_PER_W)
    items_2d = items.reshape(NW, B_PER_W)

    run = functools.partial(
        pl.kernel,
        mesh=plsc.VectorSubcoreMesh(core_axis_name="c", subcore_axis_name="s"),
        out_type=jax.ShapeDtypeStruct((BATCH,), jnp.float32),
        scratch_types=[
            pltpu.VMEM((B_PER_W,), jnp.int32),
            pltpu.VMEM((B_PER_W,), jnp.int32),
            pltpu.VMEM((K, EMBED, SLABW), jnp.float32),
            pltpu.VMEM((K, EMBED, SLABW), jnp.float32),
            pltpu.VMEM((B_PER_W,), jnp.float32),
            pltpu.SemaphoreType.DMA((K,)),
            pltpu.SemaphoreType.DMA((K,)),
        ],
        compiler_params=pltpu.CompilerParams(needs_layout_passes=False),
    )(_sc_kernel)
    return run(users_2d, items_2d, user_table.T, item_table.T)
